# Initial kernel scaffold; baseline (speedup 1.0000x reference)
import jax
import jax.numpy as jnp
from jax.experimental import pallas as pl


def kernel(f_x, v_x, quad_weights, det_A, faces, faces_to_edges, faces_to_edge_orientation):
    raise NotImplementedError("write your pallas kernel here")


# speedup vs baseline: 2.5396x; 2.5396x over previous
import functools

import jax
import jax.numpy as jnp
from jax import lax
from jax.experimental import pallas as pl
from jax.experimental.pallas import tpu as pltpu
from jax.experimental.pallas import tpu_sc as plsc

NUM_CELLS = 500000
N_QUAD = 16
N_VERTICES = 250000
N_EDGES = 750000

TC_BLOCK = 1000
TC_GRID = NUM_CELLS // TC_BLOCK

NS = 16
ROW_W = 128
CHUNK_ROWS = 16
CHUNK_W = CHUNK_ROWS * ROW_W

VFLAT = 3 * NUM_CELLS
VROWS = 12288
VPAD = VROWS * ROW_W
VROWS_TILE = VROWS // NS
VCHUNKS = VROWS_TILE // CHUNK_ROWS

EFLAT = 6 * NUM_CELLS
EROWS = 24576
EPAD = EROWS * ROW_W
EROWS_TILE = EROWS // NS
ECHUNKS = EROWS_TILE // CHUNK_ROWS

VACC = 250112
VSLICE = VACC // NS
EACC = 1500160
ESLICE = EACC // NS
ECOPY = ESLICE // 10
VCOPY = VSLICE // 2


def _integral_body(fx_ref, det_ref, o6_ref, wv_ref, we_ref, wes_ref, wf_ref,
                   vert_ref, edge_ref, face_ref):
    fx = fx_ref[...]
    det = det_ref[...]
    o6 = o6_ref[...]
    yv = jnp.dot(fx, wv_ref[...], preferred_element_type=jnp.float32)
    ye = jnp.dot(fx, we_ref[...], preferred_element_type=jnp.float32)
    ys = jnp.dot(fx, wes_ref[...], preferred_element_type=jnp.float32)
    yf = jnp.dot(fx, wf_ref[...], preferred_element_type=jnp.float32)
    vert_ref[...] = yv * det
    edge_ref[...] = (o6 * ye + (1.0 - o6) * ys) * det
    face_ref[...] = yf * det


def _integral(f_x, det2, o6, wv, we, wes, wf):
    row_spec = lambda w: pl.BlockSpec((TC_BLOCK, w), lambda i: (i, 0))
    full_spec = lambda a: pl.BlockSpec(a.shape, lambda i: (0, 0))
    return pl.pallas_call(
        _integral_body,
        grid=(TC_GRID,),
        in_specs=[row_spec(N_QUAD), row_spec(1), row_spec(6),
                  full_spec(wv), full_spec(we), full_spec(wes), full_spec(wf)],
        out_specs=[row_spec(3), row_spec(6), row_spec(1)],
        out_shape=[
            jax.ShapeDtypeStruct((NUM_CELLS, 3), jnp.float32),
            jax.ShapeDtypeStruct((NUM_CELLS, 6), jnp.float32),
            jax.ShapeDtypeStruct((NUM_CELLS, 1), jnp.float32),
        ],
    )(f_x, det2, o6, wv, we, wes, wf)


def _scatter_body(vvals, vidx, evals, eidx,
                  vout, eout, vacc, eacc, idx_buf, vbuf, cp):
    c = lax.axis_index("c")
    s = lax.axis_index("s")

    zvec = jnp.zeros((16,), jnp.float32)

    def zfill(i, carry):
        cp[pl.ds(i * 16, 16)] = zvec
        return carry
    lax.fori_loop(0, ECOPY // 16, zfill, 0)

    @pl.when(c == 0)
    def _():
        for k in range(2):
            pltpu.sync_copy(cp.at[pl.ds(0, VCOPY)],
                            vacc.at[pl.ds(s * VSLICE + k * VCOPY, VCOPY)])

    @pl.when(c != 0)
    def _():
        for k in range(10):
            pltpu.sync_copy(cp, eacc.at[pl.ds(s * ESLICE + k * ECOPY, ECOPY)])

    plsc.subcore_barrier()

    @pl.when(c == 0)
    def _():
        def chunk(t, carry):
            row0 = s * VROWS_TILE + t * CHUNK_ROWS
            pltpu.sync_copy(vidx.at[pl.ds(row0, CHUNK_ROWS)], idx_buf)
            pltpu.sync_copy(vvals.at[pl.ds(row0 * ROW_W, CHUNK_W)], vbuf)
            for j in range(CHUNK_ROWS):
                pltpu.sync_copy(vbuf.at[pl.ds(j * ROW_W, ROW_W)],
                                vacc.at[idx_buf.at[j]], add=True)
            return carry
        lax.fori_loop(0, VCHUNKS, chunk, 0)

    @pl.when(c != 0)
    def _():
        def chunk(t, carry):
            row0 = s * EROWS_TILE + t * CHUNK_ROWS
            pltpu.sync_copy(eidx.at[pl.ds(row0, CHUNK_ROWS)], idx_buf)
            pltpu.sync_copy(evals.at[pl.ds(row0 * ROW_W, CHUNK_W)], vbuf)
            for j in range(CHUNK_ROWS):
                pltpu.sync_copy(vbuf.at[pl.ds(j * ROW_W, ROW_W)],
                                eacc.at[idx_buf.at[j]], add=True)
            return carry
        lax.fori_loop(0, ECHUNKS, chunk, 0)

    plsc.subcore_barrier()

    @pl.when(c == 0)
    def _():
        for k in range(2):
            off = s * VSLICE + k * VCOPY
            pltpu.sync_copy(vacc.at[pl.ds(off, VCOPY)], cp.at[pl.ds(0, VCOPY)])
            pltpu.sync_copy(cp.at[pl.ds(0, VCOPY)], vout.at[pl.ds(off, VCOPY)])

    @pl.when(c != 0)
    def _():
        for k in range(10):
            off = s * ESLICE + k * ECOPY
            pltpu.sync_copy(eacc.at[pl.ds(off, ECOPY)], cp)
            pltpu.sync_copy(cp, eout.at[pl.ds(off, ECOPY)])


_scatter = functools.partial(
    pl.kernel,
    out_type=[
        jax.ShapeDtypeStruct((VACC,), jnp.float32),
        jax.ShapeDtypeStruct((EACC,), jnp.float32),
    ],
    mesh=plsc.VectorSubcoreMesh(core_axis_name="c", subcore_axis_name="s"),
    compiler_params=pltpu.CompilerParams(use_tc_tiling_on_sc=False),
    scratch_types=[
        pltpu.VMEM_SHARED((VACC,), jnp.float32),
        pltpu.VMEM_SHARED((EACC,), jnp.float32),
        pltpu.VMEM((CHUNK_ROWS, ROW_W), jnp.int32),
        pltpu.VMEM((CHUNK_W,), jnp.float32),
        pltpu.VMEM((ECOPY,), jnp.float32),
    ],
)(_scatter_body)


def kernel(f_x, v_x, quad_weights, det_A, faces, faces_to_edges,
           faces_to_edge_orientation):
    w = v_x * quad_weights[None, :]
    wv = w[0:3].T
    we = w[3:9].T
    wes = w[jnp.array([4, 3, 6, 5, 8, 7])].T
    wf = w[9:10].T
    det2 = det_A[:, None]
    o6 = jnp.repeat(faces_to_edge_orientation.astype(jnp.float32), 2, axis=1)

    vert_vals, edge_vals, face_dofs = _integral(f_x, det2, o6, wv, we, wes, wf)

    vvals = jnp.pad(vert_vals.reshape(VFLAT), (0, VPAD - VFLAT))
    vidx = jnp.pad(faces.reshape(VFLAT), (0, VPAD - VFLAT)).reshape(
        VROWS, ROW_W)
    evals = jnp.pad(edge_vals.reshape(EFLAT), (0, EPAD - EFLAT))
    e2 = 2 * faces_to_edges.reshape(VFLAT)
    eidx = jnp.pad(jnp.stack([e2, e2 + 1], axis=1).reshape(EFLAT),
                   (0, EPAD - EFLAT)).reshape(EROWS, ROW_W)
    vout, eout = _scatter(vvals, vidx, evals, eidx)
    vertex_dofs = vout[:N_VERTICES]
    edge_dofs = eout[:2 * N_EDGES].reshape(N_EDGES, 2)
    return (vertex_dofs, edge_dofs, face_dofs)


# speedup vs baseline: 5.7152x; 2.2504x over previous
import functools

import jax
import jax.numpy as jnp
from jax import lax
from jax.experimental import pallas as pl
from jax.experimental.pallas import tpu as pltpu
from jax.experimental.pallas import tpu_sc as plsc

NUM_CELLS = 500000
N_QUAD = 16
N_VERTICES = 250000
N_EDGES = 750000

TC_BLOCK = 1000
TC_GRID = NUM_CELLS // TC_BLOCK

NC = 2
NS = 16
NW = NC * NS
ROW_W = 128
CHUNK_ROWS = 16
CHUNK_W = CHUNK_ROWS * ROW_W

SFLAT = 9 * NUM_CELLS
SROWS = 36864
SPAD = SROWS * ROW_W
ROWS_TILE = SROWS // NW
N_CHUNKS = ROWS_TILE // CHUNK_ROWS

VACC = 250112
ACC = 1751040
ACC_TILE = ACC // NS
CP = 4560
N_CP = ACC_TILE // CP

CB_ROWS = 13680
CB_BLK = 1368


def _integral_body(fx_ref, det_ref, o9_ref, w9_ref, ws9_ref, wf_ref,
                   sv_ref, face_ref):
    fx = fx_ref[...]
    det = det_ref[...]
    o9 = o9_ref[...]
    y = jnp.dot(fx, w9_ref[...], preferred_element_type=jnp.float32)
    ys = jnp.dot(fx, ws9_ref[...], preferred_element_type=jnp.float32)
    yf = jnp.dot(fx, wf_ref[...], preferred_element_type=jnp.float32)
    sv_ref[...] = (o9 * y + (1.0 - o9) * ys) * det
    face_ref[...] = yf * det


def _integral(f_x, det2, o9, w9, ws9, wf):
    row_spec = lambda w: pl.BlockSpec((TC_BLOCK, w), lambda i: (i, 0))
    full_spec = lambda a: pl.BlockSpec(a.shape, lambda i: (0, 0))
    return pl.pallas_call(
        _integral_body,
        grid=(TC_GRID,),
        in_specs=[row_spec(N_QUAD), row_spec(1), row_spec(9),
                  full_spec(w9), full_spec(ws9), full_spec(wf)],
        out_specs=[row_spec(9), row_spec(1)],
        out_shape=[
            jax.ShapeDtypeStruct((NUM_CELLS, 9), jnp.float32),
            jax.ShapeDtypeStruct((NUM_CELLS, 1), jnp.float32),
        ],
    )(f_x, det2, o9, w9, ws9, wf)


def _scatter_body(svals, sidx, hout, acc,
                  ib0, vb0, ib1, vb1, cp, ls0, ls1, ssem, osem):
    c = lax.axis_index("c")
    s = lax.axis_index("s")
    wid = c * NS + s

    zvec = jnp.zeros((16,), jnp.float32)

    def zfill(i, carry):
        cp[pl.ds(i * 16, 16)] = zvec
        return carry
    lax.fori_loop(0, CP // 16, zfill, 0)
    for k in range(N_CP):
        pltpu.async_copy(cp, acc.at[pl.ds(s * ACC_TILE + k * CP, CP)], osem)
    for k in range(N_CP):
        pltpu.make_async_copy(
            cp, acc.at[pl.ds(s * ACC_TILE + k * CP, CP)], osem).wait()

    plsc.subcore_barrier()

    def load_start(t, ib, vb, sem):
        row0 = wid * ROWS_TILE + t * CHUNK_ROWS
        pltpu.async_copy(sidx.at[pl.ds(row0, CHUNK_ROWS)], ib, sem)
        pltpu.async_copy(svals.at[pl.ds(row0 * ROW_W, CHUNK_W)], vb, sem)

    def load_wait(ib, vb, sem):
        pltpu.make_async_copy(sidx.at[pl.ds(0, CHUNK_ROWS)], ib, sem).wait()
        pltpu.make_async_copy(svals.at[pl.ds(0, CHUNK_W)], vb, sem).wait()

    def scatter(ib, vb):
        for j in range(CHUNK_ROWS):
            pltpu.async_copy(vb.at[pl.ds(j * ROW_W, ROW_W)],
                             acc.at[ib.at[j]], ssem, add=True)
        for j in range(CHUNK_ROWS):
            pltpu.make_async_copy(vb.at[pl.ds(j * ROW_W, ROW_W)],
                                  acc.at[ib.at[j]], ssem).wait()

    load_start(0, ib0, vb0, ls0)
    load_start(1, ib1, vb1, ls1)

    def pipe(p, carry):
        t0 = 2 * p
        load_wait(ib0, vb0, ls0)
        scatter(ib0, vb0)

        @pl.when(t0 + 2 < N_CHUNKS)
        def _():
            load_start(t0 + 2, ib0, vb0, ls0)
        load_wait(ib1, vb1, ls1)
        scatter(ib1, vb1)

        @pl.when(t0 + 3 < N_CHUNKS)
        def _():
            load_start(t0 + 3, ib1, vb1, ls1)
        return carry
    lax.fori_loop(0, N_CHUNKS // 2, pipe, 0)

    plsc.subcore_barrier()

    for k in range(N_CP):
        off = s * ACC_TILE + k * CP
        pltpu.sync_copy(acc.at[pl.ds(off, CP)], cp)
        pltpu.sync_copy(cp, hout.at[pl.ds(c * ACC + off, CP)])


_scatter = functools.partial(
    pl.kernel,
    out_type=jax.ShapeDtypeStruct((NC * ACC,), jnp.float32),
    mesh=plsc.VectorSubcoreMesh(core_axis_name="c", subcore_axis_name="s"),
    compiler_params=pltpu.CompilerParams(use_tc_tiling_on_sc=False),
    scratch_types=[
        pltpu.VMEM_SHARED((ACC,), jnp.float32),
        pltpu.VMEM((CHUNK_ROWS, ROW_W), jnp.int32),
        pltpu.VMEM((CHUNK_W,), jnp.float32),
        pltpu.VMEM((CHUNK_ROWS, ROW_W), jnp.int32),
        pltpu.VMEM((CHUNK_W,), jnp.float32),
        pltpu.VMEM((CP,), jnp.float32),
        pltpu.SemaphoreType.DMA,
        pltpu.SemaphoreType.DMA,
        pltpu.SemaphoreType.DMA,
        pltpu.SemaphoreType.DMA,
    ],
)(_scatter_body)


def _combine_body(in_ref, out_ref):
    out_ref[...] = in_ref[0] + in_ref[1]


def _combine(hout2):
    return pl.pallas_call(
        _combine_body,
        grid=(CB_ROWS // CB_BLK,),
        in_specs=[pl.BlockSpec((2, CB_BLK, 128), lambda i: (0, i, 0))],
        out_specs=pl.BlockSpec((CB_BLK, 128), lambda i: (i, 0)),
        out_shape=jax.ShapeDtypeStruct((CB_ROWS, 128), jnp.float32),
    )(hout2)


def kernel(f_x, v_x, quad_weights, det_A, faces, faces_to_edges,
           faces_to_edge_orientation):
    w = v_x * quad_weights[None, :]
    w9 = w[0:9].T
    ws9 = w[jnp.array([0, 1, 2, 4, 3, 6, 5, 8, 7])].T
    wf = w[9:10].T
    det2 = det_A[:, None]
    o6 = jnp.repeat(faces_to_edge_orientation.astype(jnp.float32), 2, axis=1)
    o9 = jnp.concatenate([jnp.ones((NUM_CELLS, 3), jnp.float32), o6], axis=1)

    sv, face_dofs = _integral(f_x, det2, o9, w9, ws9, wf)

    e2 = VACC + 2 * faces_to_edges
    idx9 = jnp.concatenate(
        [faces, jnp.stack([e2, e2 + 1], axis=-1).reshape(NUM_CELLS, 6)],
        axis=1)
    svals = jnp.pad(sv.reshape(SFLAT), (0, SPAD - SFLAT))
    sidx = jnp.pad(idx9.reshape(SFLAT), (0, SPAD - SFLAT)).reshape(
        SROWS, ROW_W)

    hout = _scatter(svals, sidx)
    fin = _combine(hout.reshape(NC, CB_ROWS, 128)).reshape(ACC)

    vertex_dofs = fin[:N_VERTICES]
    edge_dofs = fin[VACC:VACC + 2 * N_EDGES].reshape(N_EDGES, 2)
    return (vertex_dofs, edge_dofs, face_dofs)
